# baseline (speedup 1.0000x reference)
import functools

import jax
import jax.numpy as jnp
from jax.experimental import pallas as pl
from jax.experimental.pallas import tpu as pltpu

_N = 1024
_H1, _F1 = 4, 64
_F2 = 64
_JB = 512


def _mm(a, b, dims):
    return jax.lax.dot_general(a, b, (dims, ((), ())),
                               preferred_element_type=jnp.float32)


def _mask_block(emb, emb_blk):
    G = _mm(emb, emb_blk, ((1,), (1,)))
    rows = jax.lax.broadcasted_iota(jnp.int32, (_N, _JB), 0)
    cols = jax.lax.broadcasted_iota(jnp.int32, (_N, _JB), 1)
    j0 = pl.program_id(0) * _JB
    return jnp.logical_or(G > 0.0, rows == cols + j0)


def _heads(h, h_blk, a_s, a_d, mask, nheads, F):
    bf16 = jnp.bfloat16
    ones_col = jnp.ones((_N, 1), dtype=bf16)
    outs = []
    for hd in range(nheads):
        hh = h[:, hd * F:(hd + 1) * F]
        hh_blk = h_blk[:, hd * F:(hd + 1) * F]
        sc = _mm(hh, a_s, ((1,), (1,)))[:, hd:hd + 1].astype(bf16)
        row = _mm(a_d, hh_blk, ((1,), (1,)))[hd:hd + 1, :].astype(bf16)
        E = sc + row
        E = jnp.maximum(E, bf16(0.2) * E)
        ex = jnp.where(mask, jnp.exp(E), bf16(0.0))
        B = jnp.concatenate([ones_col, hh.astype(bf16)], axis=1)
        oden = _mm(ex, B, ((0,), (0,)))
        outs.append(oden[:, 1:] / (oden[:, :1] + 1e-9))
    return outs


def _layer1_kernel(emb_ref, embb_ref, x_ref, xb_ref, W1_ref, a1s_ref,
                   a1d_ref, b1_ref, out_ref):
    mask = _mask_block(emb_ref[...], embb_ref[...])
    h1 = _mm(x_ref[...], W1_ref[...], ((1,), (0,)))
    h1_blk = _mm(xb_ref[...], W1_ref[...], ((1,), (0,)))
    outs = _heads(h1, h1_blk, a1s_ref[...], a1d_ref[...], mask, _H1, _F1)
    x2 = jnp.concatenate(outs, axis=1) + b1_ref[...]
    out_ref[...] = jnp.maximum(x2, 0.0)


def _layer2_kernel(emb_ref, embb_ref, x2_ref, x2b_ref, W2_ref, a2s_ref,
                   a2d_ref, b2_ref, out_ref):
    mask = _mask_block(emb_ref[...], embb_ref[...])
    h2 = _mm(x2_ref[...], W2_ref[...], ((1,), (0,)))
    h2_blk = _mm(x2b_ref[...], W2_ref[...], ((1,), (0,)))
    out = _heads(h2, h2_blk, a2s_ref[...], a2d_ref[...], mask, 1, _F2)[0]
    out_ref[...] = out + b2_ref[...]


def _full(shape):
    return pl.BlockSpec(shape, lambda g: (0, 0))


def _rowblk(cols):
    return pl.BlockSpec((_JB, cols), lambda g: (g, 0))


def kernel(x, node_embeddings, W1, a1_src, a1_dst, b1, W2, a2_src, a2_dst, b2):
    f32 = jnp.float32
    emb = node_embeddings
    grid = (_N // _JB,)
    params = pltpu.CompilerParams(dimension_semantics=("parallel",))
    x2 = pl.pallas_call(
        _layer1_kernel,
        grid=grid,
        in_specs=[_full((_N, 16)), _rowblk(16), _full((_N, 3)), _rowblk(3),
                  _full((3, _H1 * _F1)), _full((_H1, _F1)), _full((_H1, _F1)),
                  _full((1, _H1 * _F1))],
        out_specs=_rowblk(_H1 * _F1),
        out_shape=jax.ShapeDtypeStruct((_N, _H1 * _F1), f32),
        compiler_params=params,
    )(emb, emb, x, x, W1, a1_src, a1_dst, b1.reshape(1, -1))
    return pl.pallas_call(
        _layer2_kernel,
        grid=grid,
        in_specs=[_full((_N, 16)), _rowblk(16), _full((_N, _H1 * _F1)),
                  _rowblk(_H1 * _F1), _full((_H1 * _F1, _F2)),
                  _full((1, _F2)), _full((1, _F2)), _full((1, _F2))],
        out_specs=_rowblk(_F2),
        out_shape=jax.ShapeDtypeStruct((_N, _F2), f32),
        compiler_params=params,
    )(emb, emb, x2, x2, W2, a2_src, a2_dst, b2.reshape(1, -1))


# speedup vs baseline: 1.6180x; 1.6180x over previous
import jax
import jax.numpy as jnp
from jax.experimental import pallas as pl

_N = 1024
_H1, _F1 = 4, 64
_F2 = 64


def _gat_fused_kernel(emb_ref, x_ref, W1_ref, a1s_ref, a1d_ref, b1_ref,
                      W2_ref, a2s_ref, a2d_ref, b2_ref, out_ref):
    f32 = jnp.float32
    bf16 = jnp.bfloat16

    def mm(a, b, dims):
        return jax.lax.dot_general(a, b, (dims, ((), ())),
                                   preferred_element_type=f32)

    emb = emb_ref[...]
    G = mm(emb, emb, ((1,), (1,)))
    rows = jax.lax.broadcasted_iota(jnp.int32, (_N, _N), 0)
    cols = jax.lax.broadcasted_iota(jnp.int32, (_N, _N), 1)
    G = G + jnp.where(rows == cols, f32(1e30), f32(0.0))
    maskf = jnp.where(G > 0.0, f32(1.0), f32(0.0)).astype(bf16)
    ones_col = jnp.ones((_N, 1), dtype=bf16)

    def gat_layer(h, a_s, a_d, nheads, F):
        outs = []
        for hd in range(nheads):
            hh = h[:, hd * F:(hd + 1) * F]
            scd = mm(hh, a_d, ((1,), (1,)))[:, hd:hd + 1].astype(bf16)
            scr = mm(a_s, hh, ((1,), (1,)))[hd:hd + 1, :].astype(bf16)
            E = scd + scr
            E = jnp.maximum(E, bf16(0.2) * E)
            ex = jnp.exp(E) * maskf
            B = jnp.concatenate([ones_col, hh.astype(bf16)], axis=1)
            oden = mm(ex, B, ((1,), (0,)))
            outs.append(oden[:, 1:] / (oden[:, :1] + 1e-9))
        return outs

    h1 = mm(x_ref[...], W1_ref[...], ((1,), (0,)))
    x2 = jnp.concatenate(gat_layer(h1, a1s_ref[...], a1d_ref[...], _H1, _F1),
                         axis=1) + b1_ref[...]
    x2 = jnp.maximum(x2, 0.0)
    h2 = mm(x2, W2_ref[...], ((1,), (0,)))
    out2 = gat_layer(h2, a2s_ref[...], a2d_ref[...], 1, _F2)[0]
    out_ref[...] = out2 + b2_ref[...]


def kernel(x, node_embeddings, W1, a1_src, a1_dst, b1, W2, a2_src, a2_dst, b2):
    return pl.pallas_call(
        _gat_fused_kernel,
        out_shape=jax.ShapeDtypeStruct((_N, _F2), jnp.float32),
    )(node_embeddings, x, W1, a1_src, a1_dst, b1.reshape(1, -1),
      W2, a2_src, a2_dst, b2.reshape(1, -1))
